# Initial kernel scaffold; baseline (speedup 1.0000x reference)
import jax
import jax.numpy as jnp
from jax.experimental import pallas as pl


def kernel(x, edge_index, edge_weight, W0, b0, Wc, Wout, bout):
    raise NotImplementedError("write your pallas kernel here")


# speedup vs baseline: 2.4944x; 2.4944x over previous
import functools
import math

import jax
import jax.numpy as jnp
from jax import lax
from jax.experimental import pallas as pl
from jax.experimental.pallas import tpu as pltpu
from jax.experimental.pallas import tpu_sc as plsc

N = 50000
E = 800000
F_IN = 128
H = 64
HH = H // 2
C = 64
L = 4
ALPHA = 0.1
THETA = 0.5

NS = 16
EPT = E // NS
CHUNK = 128
NFULL = EPT // CHUNK
TAIL = EPT - NFULL * CHUNK
NRB_FULL = N // CHUNK
RTAIL = N - NRB_FULL * CHUNK
TAIL_TILE = NRB_FULL % NS


def _sc_propagate_body(ha, hb, src, dst, w, agg_a, agg_b,
                       src_v, dst_v, w_v, rows_v, src_t, dst_t, w_t, rows_t,
                       acc, sem):
    cid = lax.axis_index("c")
    sid = lax.axis_index("s")

    zrow = jnp.zeros((16,), jnp.float32)

    def zero_zbuf(i, _):
        rows_v[i, pl.ds(0, 16)] = zrow
        rows_v[i, pl.ds(16, 16)] = zrow
        return 0

    lax.fori_loop(0, CHUNK, zero_zbuf, 0)

    nblk = (NRB_FULL - 1 - sid) // NS + 1

    def zero_acc(i, _):
        pltpu.sync_copy(rows_v, acc.at[pl.ds((sid + i * NS) * CHUNK, CHUNK)])
        return 0

    lax.fori_loop(0, nblk, zero_acc, 0)
    if RTAIL:
        @pl.when(sid == TAIL_TILE)
        def _():
            pltpu.sync_copy(rows_v.at[pl.ds(0, RTAIL)],
                            acc.at[pl.ds(NRB_FULL * CHUNK, RTAIL)])

    plsc.subcore_barrier()

    ebase = sid * EPT

    def scale_rows(rows, wref, n):
        def body(g, _):
            base = g * 16
            for j in range(16):
                i = base + j
                w16 = plsc.load_gather(wref, [jnp.broadcast_to(i, (16,))])
                rows[i, pl.ds(0, 16)] = rows[i, pl.ds(0, 16)] * w16
                rows[i, pl.ds(16, 16)] = rows[i, pl.ds(16, 16)] * w16
            return 0

        lax.fori_loop(0, n // 16, body, 0)

    def run(h_half):
        def chunk_body(g, _):
            b = ebase + g * CHUNK
            pltpu.sync_copy(src.at[pl.ds(b, CHUNK)], src_v)
            pltpu.sync_copy(dst.at[pl.ds(b, CHUNK)], dst_v)
            pltpu.sync_copy(w.at[pl.ds(b, CHUNK)], w_v)
            pltpu.async_copy(h_half.at[src_v], rows_v, sem).wait()
            scale_rows(rows_v, w_v, CHUNK)
            pltpu.sync_copy(rows_v, acc.at[dst_v], add=True)
            return 0

        lax.fori_loop(0, NFULL, chunk_body, 0)
        if TAIL:
            b = ebase + NFULL * CHUNK
            pltpu.sync_copy(src.at[pl.ds(b, TAIL)], src_t)
            pltpu.sync_copy(dst.at[pl.ds(b, TAIL)], dst_t)
            pltpu.sync_copy(w.at[pl.ds(b, TAIL)], w_t)
            pltpu.async_copy(h_half.at[src_t], rows_t, sem).wait()
            scale_rows(rows_t, w_t, TAIL)
            pltpu.sync_copy(rows_t, acc.at[dst_t], add=True)

    @pl.when(cid == 0)
    def _():
        run(ha)

    @pl.when(cid == 1)
    def _():
        run(hb)

    plsc.subcore_barrier()

    def copy_out(out):
        def body(i, _):
            rr = (sid + i * NS) * CHUNK
            pltpu.sync_copy(acc.at[pl.ds(rr, CHUNK)], rows_v)
            pltpu.sync_copy(rows_v, out.at[pl.ds(rr, CHUNK)])
            return 0

        lax.fori_loop(0, nblk, body, 0)
        if RTAIL:
            @pl.when(sid == TAIL_TILE)
            def _():
                rr = NRB_FULL * CHUNK
                pltpu.sync_copy(acc.at[pl.ds(rr, RTAIL)],
                                rows_t.at[pl.ds(0, RTAIL)])
                pltpu.sync_copy(rows_t.at[pl.ds(0, RTAIL)],
                                out.at[pl.ds(rr, RTAIL)])

    @pl.when(cid == 0)
    def _():
        copy_out(agg_a)

    @pl.when(cid == 1)
    def _():
        copy_out(agg_b)


def _sc_propagate(ha, hb, src, dst, w):
    mesh = plsc.VectorSubcoreMesh(core_axis_name="c", subcore_axis_name="s")
    f32 = jnp.float32
    return pl.kernel(
        _sc_propagate_body,
        out_type=[jax.ShapeDtypeStruct((N, HH), f32),
                  jax.ShapeDtypeStruct((N, HH), f32)],
        mesh=mesh,
        scratch_types=[
            pltpu.VMEM((CHUNK,), jnp.int32),
            pltpu.VMEM((CHUNK,), jnp.int32),
            pltpu.VMEM((CHUNK,), f32),
            pltpu.VMEM((CHUNK, HH), f32),
            pltpu.VMEM((TAIL,), jnp.int32),
            pltpu.VMEM((TAIL,), jnp.int32),
            pltpu.VMEM((TAIL,), f32),
            pltpu.VMEM((TAIL, HH), f32),
            pltpu.VMEM_SHARED((N, HH), f32),
            pltpu.SemaphoreType.DMA,
        ],
        compiler_params=pltpu.CompilerParams(needs_layout_passes=False,
                                             use_tc_tiling_on_sc=False),
    )(ha, hb, src, dst, w)


BN = 1024


def _proj_body(x_ref, w_ref, b_ref, ha_ref, hb_ref):
    h = jnp.dot(x_ref[...], w_ref[...], preferred_element_type=jnp.float32)
    h = jnp.maximum(h + b_ref[...], 0.0)
    ha_ref[...] = h[:, :HH]
    hb_ref[...] = h[:, HH:]


def _tc_project(x, w0t, b0):
    grid = (pl.cdiv(N, BN),)
    return pl.pallas_call(
        _proj_body,
        grid=grid,
        in_specs=[
            pl.BlockSpec((BN, F_IN), lambda i: (i, 0)),
            pl.BlockSpec((F_IN, H), lambda i: (0, 0)),
            pl.BlockSpec((1, H), lambda i: (0, 0)),
        ],
        out_specs=[pl.BlockSpec((BN, HH), lambda i: (i, 0)),
                   pl.BlockSpec((BN, HH), lambda i: (i, 0))],
        out_shape=[jax.ShapeDtypeStruct((N, HH), jnp.float32),
                   jax.ShapeDtypeStruct((N, HH), jnp.float32)],
    )(x, w0t, b0)


def _layer_body(aa_ref, ab_ref, xa_ref, xb_ref, m_ref, ha_ref, hb_ref):
    u = jnp.concatenate(
        [(1.0 - ALPHA) * aa_ref[...] + ALPHA * xa_ref[...],
         (1.0 - ALPHA) * ab_ref[...] + ALPHA * xb_ref[...]], axis=1)
    o = jnp.maximum(jnp.dot(u, m_ref[...], preferred_element_type=jnp.float32),
                    0.0)
    ha_ref[...] = o[:, :HH]
    hb_ref[...] = o[:, HH:]


def _tc_layer(agg_a, agg_b, x0a, x0b, m):
    grid = (pl.cdiv(N, BN),)
    half = pl.BlockSpec((BN, HH), lambda i: (i, 0))
    return pl.pallas_call(
        _layer_body,
        grid=grid,
        in_specs=[half, half, half, half,
                  pl.BlockSpec((H, H), lambda i: (0, 0))],
        out_specs=[half, half],
        out_shape=[jax.ShapeDtypeStruct((N, HH), jnp.float32),
                   jax.ShapeDtypeStruct((N, HH), jnp.float32)],
    )(agg_a, agg_b, x0a, x0b, m)


def _final_body(aa_ref, ab_ref, xa_ref, xb_ref, m_ref, wout_ref, bout_ref,
                out_ref):
    u = jnp.concatenate(
        [(1.0 - ALPHA) * aa_ref[...] + ALPHA * xa_ref[...],
         (1.0 - ALPHA) * ab_ref[...] + ALPHA * xb_ref[...]], axis=1)
    h = jnp.maximum(jnp.dot(u, m_ref[...], preferred_element_type=jnp.float32),
                    0.0)
    logits = jnp.dot(h, wout_ref[...], preferred_element_type=jnp.float32)
    logits = logits + bout_ref[...]
    mx = jnp.max(logits, axis=1, keepdims=True)
    ex = jnp.exp(logits - mx)
    lse = jnp.log(jnp.sum(ex, axis=1, keepdims=True))
    out_ref[...] = logits - mx - lse


def _tc_final(agg_a, agg_b, x0a, x0b, m, woutt, bout):
    grid = (pl.cdiv(N, BN),)
    half = pl.BlockSpec((BN, HH), lambda i: (i, 0))
    return pl.pallas_call(
        _final_body,
        grid=grid,
        in_specs=[half, half, half, half,
                  pl.BlockSpec((H, H), lambda i: (0, 0)),
                  pl.BlockSpec((H, C), lambda i: (0, 0)),
                  pl.BlockSpec((1, C), lambda i: (0, 0))],
        out_specs=pl.BlockSpec((BN, C), lambda i: (i, 0)),
        out_shape=jax.ShapeDtypeStruct((N, C), jnp.float32),
    )(agg_a, agg_b, x0a, x0b, m, woutt, bout)


def kernel(x, edge_index, edge_weight, W0, b0, Wc, Wout, bout):
    src = edge_index[0].astype(jnp.int32)
    dst = edge_index[1].astype(jnp.int32)
    w = edge_weight.astype(jnp.float32)

    eye = jnp.eye(H, dtype=jnp.float32)
    betas = [math.log(THETA / (l + 1) + 1.0) for l in range(L)]
    ms = [(1.0 - b) * eye + b * Wc[l] for l, b in enumerate(betas)]

    ha, hb = _tc_project(x, W0.T, b0.reshape(1, H))
    x0a, x0b = ha, hb
    for l in range(L - 1):
        agg_a, agg_b = _sc_propagate(ha, hb, src, dst, w)
        ha, hb = _tc_layer(agg_a, agg_b, x0a, x0b, ms[l])
    agg_a, agg_b = _sc_propagate(ha, hb, src, dst, w)
    return _tc_final(agg_a, agg_b, x0a, x0b, ms[L - 1], Wout.T,
                     bout.reshape(1, C))


# speedup vs baseline: 5.1408x; 2.0609x over previous
import functools
import math

import jax
import jax.numpy as jnp
from jax import lax
from jax.experimental import pallas as pl
from jax.experimental.pallas import tpu as pltpu
from jax.experimental.pallas import tpu_sc as plsc

N = 50000
E = 800000
F_IN = 128
H = 64
HH = H // 2
C = 64
L = 4
ALPHA = 0.1
THETA = 0.5

NS = 16
EPT = E // NS
CHUNK = 128
NFULL = EPT // CHUNK
TAIL = EPT - NFULL * CHUNK
NRB_FULL = N // CHUNK
RTAIL = N - NRB_FULL * CHUNK
TAIL_TILE = NRB_FULL % NS


def _sc_propagate_body(ha, hb, src, dst, w, agg_a, agg_b,
                       srcs, dsts, ws, rows, src_t, dst_t, w_t, rows_t,
                       acc, sg0, sg1, sg2, ss0, ss1, ss2, si0, si1, si2,
                       sem):
    sem_g = (sg0, sg1, sg2)
    sem_s = (ss0, ss1, ss2)
    sem_i = (si0, si1, si2)
    cid = lax.axis_index("c")
    sid = lax.axis_index("s")

    rows_v = rows.at[0]
    zrow = jnp.zeros((16,), jnp.float32)

    def zero_zbuf(i, _):
        rows[0, i, pl.ds(0, 16)] = zrow
        rows[0, i, pl.ds(16, 16)] = zrow
        return 0

    lax.fori_loop(0, CHUNK, zero_zbuf, 0)

    nblk = (NRB_FULL - 1 - sid) // NS + 1

    def zero_acc(i, _):
        pltpu.sync_copy(rows_v, acc.at[pl.ds((sid + i * NS) * CHUNK, CHUNK)])
        return 0

    lax.fori_loop(0, nblk, zero_acc, 0)
    if RTAIL:
        @pl.when(sid == TAIL_TILE)
        def _():
            pltpu.sync_copy(rows_v.at[pl.ds(0, RTAIL)],
                            acc.at[pl.ds(NRB_FULL * CHUNK, RTAIL)])

    plsc.subcore_barrier()

    ebase = sid * EPT

    def scale_rows(rref, wref, n):
        def body(g, _):
            base = g * 16
            for j in range(16):
                i = base + j
                w16 = plsc.load_gather(wref, [jnp.broadcast_to(i, (16,))])
                rref[i, pl.ds(0, 16)] = rref[i, pl.ds(0, 16)] * w16
                rref[i, pl.ds(16, 16)] = rref[i, pl.ds(16, 16)] * w16
            return 0

        lax.fori_loop(0, n // 16, body, 0)

    def run(h_half, out_unused=None):
        def idx_issue(g, p):
            b = ebase + g * CHUNK
            pltpu.async_copy(src.at[pl.ds(b, CHUNK)], srcs.at[p], sem_i[p])
            pltpu.async_copy(dst.at[pl.ds(b, CHUNK)], dsts.at[p], sem_i[p])
            pltpu.async_copy(w.at[pl.ds(b, CHUNK)], ws.at[p], sem_i[p])

        def idx_wait(p):
            pltpu.make_async_copy(src.at[pl.ds(0, CHUNK)], srcs.at[p], sem_i[p]).wait()
            pltpu.make_async_copy(dst.at[pl.ds(0, CHUNK)], dsts.at[p], sem_i[p]).wait()
            pltpu.make_async_copy(w.at[pl.ds(0, CHUNK)], ws.at[p], sem_i[p]).wait()

        def gather_issue(p):
            pltpu.async_copy(h_half.at[srcs.at[p]], rows.at[p], sem_g[p])

        def gather_wait(p):
            pltpu.make_async_copy(h_half.at[srcs.at[p]], rows.at[p], sem_g[p]).wait()

        def scatter_issue(p):
            pltpu.async_copy(rows.at[p], acc.at[dsts.at[p]], sem_s[p], add=True)

        def scatter_wait(p):
            pltpu.make_async_copy(rows.at[p], acc.at[dsts.at[p]], sem_s[p]).wait()

        idx_issue(0, 0)
        idx_issue(1, 1)
        idx_wait(0)
        gather_issue(0)

        def body3(k, _):
            for p in range(3):
                g = 3 * k + p
                qn = (p + 1) % 3
                qp = (p + 2) % 3
                gather_wait(p)

                @pl.when(g + 1 < NFULL)
                def _():
                    idx_wait(qn)
                    gather_issue(qn)

                scale_rows(rows.at[p], ws.at[p], CHUNK)
                scatter_issue(p)

                @pl.when(g >= 1)
                def _():
                    scatter_wait(qp)

                @pl.when(g + 2 < NFULL)
                def _():
                    idx_issue(g + 2, qp)
            return 0

        lax.fori_loop(0, NFULL // 3, body3, 0)
        scatter_wait((NFULL - 1) % 3)
        if TAIL:
            b = ebase + NFULL * CHUNK
            pltpu.sync_copy(src.at[pl.ds(b, TAIL)], src_t)
            pltpu.sync_copy(dst.at[pl.ds(b, TAIL)], dst_t)
            pltpu.sync_copy(w.at[pl.ds(b, TAIL)], w_t)
            pltpu.async_copy(h_half.at[src_t], rows_t, sem).wait()
            scale_rows(rows_t, w_t, TAIL)
            pltpu.sync_copy(rows_t, acc.at[dst_t], add=True)

    @pl.when(cid == 0)
    def _():
        run(ha)

    @pl.when(cid == 1)
    def _():
        run(hb)

    plsc.subcore_barrier()

    def copy_out(out):
        def body(i, _):
            rr = (sid + i * NS) * CHUNK
            pltpu.sync_copy(acc.at[pl.ds(rr, CHUNK)], rows_v)
            pltpu.sync_copy(rows_v, out.at[pl.ds(rr, CHUNK)])
            return 0

        lax.fori_loop(0, nblk, body, 0)
        if RTAIL:
            @pl.when(sid == TAIL_TILE)
            def _():
                rr = NRB_FULL * CHUNK
                pltpu.sync_copy(acc.at[pl.ds(rr, RTAIL)],
                                rows_t.at[pl.ds(0, RTAIL)])
                pltpu.sync_copy(rows_t.at[pl.ds(0, RTAIL)],
                                out.at[pl.ds(rr, RTAIL)])

    @pl.when(cid == 0)
    def _():
        copy_out(agg_a)

    @pl.when(cid == 1)
    def _():
        copy_out(agg_b)


def _sc_propagate(ha, hb, src, dst, w):
    mesh = plsc.VectorSubcoreMesh(core_axis_name="c", subcore_axis_name="s")
    f32 = jnp.float32
    return pl.kernel(
        _sc_propagate_body,
        out_type=[jax.ShapeDtypeStruct((N, HH), f32),
                  jax.ShapeDtypeStruct((N, HH), f32)],
        mesh=mesh,
        scratch_types=[
            pltpu.VMEM((3, CHUNK), jnp.int32),
            pltpu.VMEM((3, CHUNK), jnp.int32),
            pltpu.VMEM((3, CHUNK), f32),
            pltpu.VMEM((3, CHUNK, HH), f32),
            pltpu.VMEM((TAIL,), jnp.int32),
            pltpu.VMEM((TAIL,), jnp.int32),
            pltpu.VMEM((TAIL,), f32),
            pltpu.VMEM((TAIL, HH), f32),
            pltpu.VMEM_SHARED((N, HH), f32),
            pltpu.SemaphoreType.DMA,
            pltpu.SemaphoreType.DMA,
            pltpu.SemaphoreType.DMA,
            pltpu.SemaphoreType.DMA,
            pltpu.SemaphoreType.DMA,
            pltpu.SemaphoreType.DMA,
            pltpu.SemaphoreType.DMA,
            pltpu.SemaphoreType.DMA,
            pltpu.SemaphoreType.DMA,
            pltpu.SemaphoreType.DMA,
        ],
        compiler_params=pltpu.CompilerParams(needs_layout_passes=False,
                                             use_tc_tiling_on_sc=False),
    )(ha, hb, src, dst, w)


BN = 1024


def _proj_body(x_ref, w_ref, b_ref, ha_ref, hb_ref):
    h = jnp.dot(x_ref[...], w_ref[...], preferred_element_type=jnp.float32)
    h = jnp.maximum(h + b_ref[...], 0.0)
    ha_ref[...] = h[:, :HH]
    hb_ref[...] = h[:, HH:]


def _tc_project(x, w0t, b0):
    grid = (pl.cdiv(N, BN),)
    return pl.pallas_call(
        _proj_body,
        grid=grid,
        in_specs=[
            pl.BlockSpec((BN, F_IN), lambda i: (i, 0)),
            pl.BlockSpec((F_IN, H), lambda i: (0, 0)),
            pl.BlockSpec((1, H), lambda i: (0, 0)),
        ],
        out_specs=[pl.BlockSpec((BN, HH), lambda i: (i, 0)),
                   pl.BlockSpec((BN, HH), lambda i: (i, 0))],
        out_shape=[jax.ShapeDtypeStruct((N, HH), jnp.float32),
                   jax.ShapeDtypeStruct((N, HH), jnp.float32)],
    )(x, w0t, b0)


def _layer_body(aa_ref, ab_ref, xa_ref, xb_ref, m_ref, ha_ref, hb_ref):
    u = jnp.concatenate(
        [(1.0 - ALPHA) * aa_ref[...] + ALPHA * xa_ref[...],
         (1.0 - ALPHA) * ab_ref[...] + ALPHA * xb_ref[...]], axis=1)
    o = jnp.maximum(jnp.dot(u, m_ref[...], preferred_element_type=jnp.float32),
                    0.0)
    ha_ref[...] = o[:, :HH]
    hb_ref[...] = o[:, HH:]


def _tc_layer(agg_a, agg_b, x0a, x0b, m):
    grid = (pl.cdiv(N, BN),)
    half = pl.BlockSpec((BN, HH), lambda i: (i, 0))
    return pl.pallas_call(
        _layer_body,
        grid=grid,
        in_specs=[half, half, half, half,
                  pl.BlockSpec((H, H), lambda i: (0, 0))],
        out_specs=[half, half],
        out_shape=[jax.ShapeDtypeStruct((N, HH), jnp.float32),
                   jax.ShapeDtypeStruct((N, HH), jnp.float32)],
    )(agg_a, agg_b, x0a, x0b, m)


def _final_body(aa_ref, ab_ref, xa_ref, xb_ref, m_ref, wout_ref, bout_ref,
                out_ref):
    u = jnp.concatenate(
        [(1.0 - ALPHA) * aa_ref[...] + ALPHA * xa_ref[...],
         (1.0 - ALPHA) * ab_ref[...] + ALPHA * xb_ref[...]], axis=1)
    h = jnp.maximum(jnp.dot(u, m_ref[...], preferred_element_type=jnp.float32),
                    0.0)
    logits = jnp.dot(h, wout_ref[...], preferred_element_type=jnp.float32)
    logits = logits + bout_ref[...]
    mx = jnp.max(logits, axis=1, keepdims=True)
    ex = jnp.exp(logits - mx)
    lse = jnp.log(jnp.sum(ex, axis=1, keepdims=True))
    out_ref[...] = logits - mx - lse


def _tc_final(agg_a, agg_b, x0a, x0b, m, woutt, bout):
    grid = (pl.cdiv(N, BN),)
    half = pl.BlockSpec((BN, HH), lambda i: (i, 0))
    return pl.pallas_call(
        _final_body,
        grid=grid,
        in_specs=[half, half, half, half,
                  pl.BlockSpec((H, H), lambda i: (0, 0)),
                  pl.BlockSpec((H, C), lambda i: (0, 0)),
                  pl.BlockSpec((1, C), lambda i: (0, 0))],
        out_specs=pl.BlockSpec((BN, C), lambda i: (i, 0)),
        out_shape=jax.ShapeDtypeStruct((N, C), jnp.float32),
    )(agg_a, agg_b, x0a, x0b, m, woutt, bout)


def kernel(x, edge_index, edge_weight, W0, b0, Wc, Wout, bout):
    src = edge_index[0].astype(jnp.int32)
    dst = edge_index[1].astype(jnp.int32)
    w = edge_weight.astype(jnp.float32)

    eye = jnp.eye(H, dtype=jnp.float32)
    betas = [math.log(THETA / (l + 1) + 1.0) for l in range(L)]
    ms = [(1.0 - b) * eye + b * Wc[l] for l, b in enumerate(betas)]

    ha, hb = _tc_project(x, W0.T, b0.reshape(1, H))
    x0a, x0b = ha, hb
    for l in range(L - 1):
        agg_a, agg_b = _sc_propagate(ha, hb, src, dst, w)
        ha, hb = _tc_layer(agg_a, agg_b, x0a, x0b, ms[l])
    agg_a, agg_b = _sc_propagate(ha, hb, src, dst, w)
    return _tc_final(agg_a, agg_b, x0a, x0b, ms[L - 1], Wout.T,
                     bout.reshape(1, C))
